# baseline (speedup 1.0000x reference)
import functools

import jax
import jax.numpy as jnp
from jax.experimental import pallas as pl
from jax.experimental.pallas import tpu as pltpu


def _bilstm_kernel(xf_ref, xb_ref, wif_ref, wib_ref, whf_ref, whb_ref,
                   bf_ref, bb_ref, len_ref, out_ref,
                   xgf_ref, xgb_ref, state_ref, *, TB, NT, B, H):
    i = pl.program_id(0)

    xf = xf_ref[:].reshape(TB * B, -1)
    xgf_ref[:] = (
        jnp.dot(xf, wif_ref[:], preferred_element_type=jnp.float32) + bf_ref[:]
    )
    xb = xb_ref[:].reshape(TB * B, -1)
    xgb_ref[:] = (
        jnp.dot(xb, wib_ref[:], preferred_element_type=jnp.float32) + bb_ref[:]
    )

    @pl.when(i == 0)
    def _():
        state_ref[:] = jnp.zeros_like(state_ref)

    h_f = state_ref[:, 0:H]
    c_f = state_ref[:, H:2 * H]
    h_b = state_ref[:, 2 * H:3 * H]
    c_b = state_ref[:, 3 * H:4 * H]

    lens = len_ref[:]
    whf = whf_ref[:]
    whb = whb_ref[:]
    t0_f = i * TB
    t0_b = (NT - 1 - i) * TB

    Bh = B // 2

    def substep(s0, row, xg_ref, wh, lens_half, h, c):
        g = xg_ref[pl.ds(s0 * B + row, Bh), :] + jnp.dot(
            h.astype(jnp.bfloat16), wh, preferred_element_type=jnp.float32)
        c_new = (jax.nn.sigmoid(g[:, H:2 * H]) * c
                 + jax.nn.sigmoid(g[:, 0:H]) * jnp.tanh(g[:, 2 * H:3 * H]))
        h_new = jax.nn.sigmoid(g[:, 3 * H:4 * H]) * jnp.tanh(c_new)
        return h_new, c_new

    lens_lo = lens[0:Bh]
    lens_hi = lens[Bh:B]

    def step(k, carry):
        hf1, cf1, hf2, cf2, hb1, cb1, hb2, cb2 = carry
        s = k * 2
        for su in (s, s + 1):
            sb = TB - 1 - su
            nf1 = substep(su, 0, xgf_ref, whf, lens_lo, hf1, cf1)
            nf2 = substep(su, Bh, xgf_ref, whf, lens_hi, hf2, cf2)
            nb1 = substep(sb, 0, xgb_ref, whb, lens_lo, hb1, cb1)
            nb2 = substep(sb, Bh, xgb_ref, whb, lens_hi, hb2, cb2)
            mf = lens_lo > (t0_f + su)
            mg = lens_hi > (t0_f + su)
            mb = lens_lo > (t0_b + sb)
            mc = lens_hi > (t0_b + sb)
            hf1 = jnp.where(mf, nf1[0], hf1)
            cf1 = jnp.where(mf, nf1[1], cf1)
            hf2 = jnp.where(mg, nf2[0], hf2)
            cf2 = jnp.where(mg, nf2[1], cf2)
            hb1 = jnp.where(mb, nb1[0], hb1)
            cb1 = jnp.where(mb, nb1[1], cb1)
            hb2 = jnp.where(mc, nb2[0], hb2)
            cb2 = jnp.where(mc, nb2[1], cb2)
        return (hf1, cf1, hf2, cf2, hb1, cb1, hb2, cb2)

    hf1, cf1, hf2, cf2, hb1, cb1, hb2, cb2 = jax.lax.fori_loop(
        0, TB // 2, step,
        (h_f[0:Bh], c_f[0:Bh], h_f[Bh:B], c_f[Bh:B],
         h_b[0:Bh], c_b[0:Bh], h_b[Bh:B], c_b[Bh:B]))
    h_f = jnp.concatenate([hf1, hf2], axis=0)
    c_f = jnp.concatenate([cf1, cf2], axis=0)
    h_b = jnp.concatenate([hb1, hb2], axis=0)
    c_b = jnp.concatenate([cb1, cb2], axis=0)
    state_ref[:] = jnp.concatenate([h_f, c_f, h_b, c_b], axis=1)

    @pl.when(i == NT - 1)
    def _():
        out_ref[:] = state_ref[:]


def kernel(x, lengths, w_ih_f, w_hh_f, b_ih_f, b_hh_f,
           w_ih_b, w_hh_b, b_ih_b, b_hh_b):
    B, T, D = x.shape
    H = w_hh_f.shape[1]
    TB = 64
    NT = T // TB

    xT = jnp.transpose(x.astype(jnp.bfloat16), (1, 0, 2))
    wif = w_ih_f.T.astype(jnp.bfloat16)
    wib = w_ih_b.T.astype(jnp.bfloat16)
    whf = w_hh_f.T.astype(jnp.bfloat16)
    whb = w_hh_b.T.astype(jnp.bfloat16)
    bf = (b_ih_f + b_hh_f).reshape(1, 4 * H)
    bb = (b_ih_b + b_hh_b).reshape(1, 4 * H)
    lenb = jnp.broadcast_to(lengths.astype(jnp.int32)[:, None], (B, H))

    return pl.pallas_call(
        functools.partial(_bilstm_kernel, TB=TB, NT=NT, B=B, H=H),
        grid=(NT,),
        in_specs=[
            pl.BlockSpec((TB, B, D), lambda i: (i, 0, 0)),
            pl.BlockSpec((TB, B, D), lambda i: (NT - 1 - i, 0, 0)),
            pl.BlockSpec((D, 4 * H), lambda i: (0, 0)),
            pl.BlockSpec((D, 4 * H), lambda i: (0, 0)),
            pl.BlockSpec((H, 4 * H), lambda i: (0, 0)),
            pl.BlockSpec((H, 4 * H), lambda i: (0, 0)),
            pl.BlockSpec((1, 4 * H), lambda i: (0, 0)),
            pl.BlockSpec((1, 4 * H), lambda i: (0, 0)),
            pl.BlockSpec((B, H), lambda i: (0, 0)),
        ],
        out_specs=pl.BlockSpec((B, 4 * H), lambda i: (0, 0)),
        out_shape=jax.ShapeDtypeStruct((B, 4 * H), jnp.float32),
        scratch_shapes=[
            pltpu.VMEM((TB * B, 4 * H), jnp.float32),
            pltpu.VMEM((TB * B, 4 * H), jnp.float32),
            pltpu.VMEM((B, 4 * H), jnp.float32),
        ],
        compiler_params=pltpu.CompilerParams(
            dimension_semantics=("arbitrary",),
        ),
    )(xT, xT, wif, wib, whf, whb, bf, bb, lenb)


# speedup vs baseline: 1.5353x; 1.5353x over previous
import functools

import jax
import jax.numpy as jnp
from jax.experimental import pallas as pl
from jax.experimental.pallas import tpu as pltpu


def _bilstm_kernel(xf_ref, xb_ref, wif_ref, wib_ref, whf_ref, whb_ref,
                   bf_ref, bb_ref, len_ref, out_ref,
                   xgf_ref, xgb_ref, state_ref, *, TB, NT, B, H):
    i = pl.program_id(0)

    xf = xf_ref[:].reshape(TB * B, -1)
    xgf_ref[:] = (
        jnp.dot(xf, wif_ref[:], preferred_element_type=jnp.float32) + bf_ref[:]
    )
    xb = xb_ref[:].reshape(TB * B, -1)
    xgb_ref[:] = (
        jnp.dot(xb, wib_ref[:], preferred_element_type=jnp.float32) + bb_ref[:]
    )

    @pl.when(i == 0)
    def _():
        state_ref[:] = jnp.zeros_like(state_ref)

    h_f = state_ref[:, 0:H]
    c_f = state_ref[:, H:2 * H]
    h_b = state_ref[:, 2 * H:3 * H]
    c_b = state_ref[:, 3 * H:4 * H]

    lens = len_ref[:]
    whf = whf_ref[:]
    whb = whb_ref[:]
    t0_f = i * TB
    t0_b = (NT - 1 - i) * TB

    UNROLL = 4

    def substep_f(s, h_f, c_f):
        gf = xgf_ref[pl.ds(s * B, B), :] + jnp.dot(
            h_f.astype(jnp.bfloat16), whf, preferred_element_type=jnp.float32)
        cf_new = (jax.nn.sigmoid(gf[:, H:2 * H]) * c_f
                  + jax.nn.sigmoid(gf[:, 0:H]) * jnp.tanh(gf[:, 2 * H:3 * H]))
        hf_new = jax.nn.sigmoid(gf[:, 3 * H:4 * H]) * jnp.tanh(cf_new)
        m_f = lens > (t0_f + s)
        return jnp.where(m_f, hf_new, h_f), jnp.where(m_f, cf_new, c_f)

    def substep_b(s, h_b, c_b):
        sb = TB - 1 - s
        gb = xgb_ref[pl.ds(sb * B, B), :] + jnp.dot(
            h_b.astype(jnp.bfloat16), whb, preferred_element_type=jnp.float32)
        cb_new = (jax.nn.sigmoid(gb[:, H:2 * H]) * c_b
                  + jax.nn.sigmoid(gb[:, 0:H]) * jnp.tanh(gb[:, 2 * H:3 * H]))
        hb_new = jax.nn.sigmoid(gb[:, 3 * H:4 * H]) * jnp.tanh(cb_new)
        m_b = lens > (t0_b + sb)
        return jnp.where(m_b, hb_new, h_b), jnp.where(m_b, cb_new, c_b)

    def step(k, carry):
        h_f, c_f, h_b, c_b = carry
        s = k * UNROLL
        for u in range(UNROLL):
            h_f, c_f = substep_f(s + u, h_f, c_f)
            h_b, c_b = substep_b(s + u, h_b, c_b)
        return (h_f, c_f, h_b, c_b)

    h_f, c_f, h_b, c_b = jax.lax.fori_loop(
        0, TB // UNROLL, step, (h_f, c_f, h_b, c_b))
    state_ref[:] = jnp.concatenate([h_f, c_f, h_b, c_b], axis=1)

    @pl.when(i == NT - 1)
    def _():
        out_ref[:] = state_ref[:]


def kernel(x, lengths, w_ih_f, w_hh_f, b_ih_f, b_hh_f,
           w_ih_b, w_hh_b, b_ih_b, b_hh_b):
    B, T, D = x.shape
    H = w_hh_f.shape[1]
    TB = 64
    NT = T // TB

    xT = jnp.transpose(x.astype(jnp.bfloat16), (1, 0, 2))
    wif = w_ih_f.T.astype(jnp.bfloat16)
    wib = w_ih_b.T.astype(jnp.bfloat16)
    whf = w_hh_f.T.astype(jnp.bfloat16)
    whb = w_hh_b.T.astype(jnp.bfloat16)
    bf = (b_ih_f + b_hh_f).reshape(1, 4 * H)
    bb = (b_ih_b + b_hh_b).reshape(1, 4 * H)
    lenb = jnp.broadcast_to(lengths.astype(jnp.int32)[:, None], (B, H))

    return pl.pallas_call(
        functools.partial(_bilstm_kernel, TB=TB, NT=NT, B=B, H=H),
        grid=(NT,),
        in_specs=[
            pl.BlockSpec((TB, B, D), lambda i: (i, 0, 0)),
            pl.BlockSpec((TB, B, D), lambda i: (NT - 1 - i, 0, 0)),
            pl.BlockSpec((D, 4 * H), lambda i: (0, 0)),
            pl.BlockSpec((D, 4 * H), lambda i: (0, 0)),
            pl.BlockSpec((H, 4 * H), lambda i: (0, 0)),
            pl.BlockSpec((H, 4 * H), lambda i: (0, 0)),
            pl.BlockSpec((1, 4 * H), lambda i: (0, 0)),
            pl.BlockSpec((1, 4 * H), lambda i: (0, 0)),
            pl.BlockSpec((B, H), lambda i: (0, 0)),
        ],
        out_specs=pl.BlockSpec((B, 4 * H), lambda i: (0, 0)),
        out_shape=jax.ShapeDtypeStruct((B, 4 * H), jnp.float32),
        scratch_shapes=[
            pltpu.VMEM((TB * B, 4 * H), jnp.float32),
            pltpu.VMEM((TB * B, 4 * H), jnp.float32),
            pltpu.VMEM((B, 4 * H), jnp.float32),
        ],
        compiler_params=pltpu.CompilerParams(
            dimension_semantics=("arbitrary",),
        ),
    )(xT, xT, wif, wib, whf, whb, bf, bb, lenb)


# speedup vs baseline: 1.6247x; 1.0582x over previous
import functools

import jax
import jax.numpy as jnp
from jax.experimental import pallas as pl
from jax.experimental.pallas import tpu as pltpu


def _bilstm_kernel(xf_ref, xb_ref, wif_ref, wib_ref, whf_ref, whb_ref,
                   bf_ref, bb_ref, len_ref, out_ref,
                   xgf_ref, xgb_ref, state_ref, *, TB, NT, B, H):
    i = pl.program_id(0)

    xf = xf_ref[:].reshape(TB * B, -1)
    xgf_ref[:] = (
        jnp.dot(xf, wif_ref[:], preferred_element_type=jnp.float32) + bf_ref[:]
    )
    xb = xb_ref[:].reshape(TB * B, -1)
    xgb_ref[:] = (
        jnp.dot(xb, wib_ref[:], preferred_element_type=jnp.float32) + bb_ref[:]
    )

    @pl.when(i == 0)
    def _():
        state_ref[:] = jnp.zeros_like(state_ref)

    h_f = state_ref[:, 0:H]
    c_f = state_ref[:, H:2 * H]
    h_b = state_ref[:, 2 * H:3 * H]
    c_b = state_ref[:, 3 * H:4 * H]

    lens = len_ref[:]
    whf = whf_ref[:]
    whb = whb_ref[:]
    t0_f = i * TB
    t0_b = (NT - 1 - i) * TB

    UNROLL = 8

    def substep_f(s, h_f, c_f):
        gf = xgf_ref[pl.ds(s * B, B), :] + jnp.dot(
            h_f.astype(jnp.bfloat16), whf, preferred_element_type=jnp.float32)
        cf_new = (jax.nn.sigmoid(gf[:, H:2 * H]) * c_f
                  + jax.nn.sigmoid(gf[:, 0:H]) * jnp.tanh(gf[:, 2 * H:3 * H]))
        hf_new = jax.nn.sigmoid(gf[:, 3 * H:4 * H]) * jnp.tanh(cf_new)
        m_f = lens > (t0_f + s)
        return jnp.where(m_f, hf_new, h_f), jnp.where(m_f, cf_new, c_f)

    def substep_b(s, h_b, c_b):
        sb = TB - 1 - s
        gb = xgb_ref[pl.ds(sb * B, B), :] + jnp.dot(
            h_b.astype(jnp.bfloat16), whb, preferred_element_type=jnp.float32)
        cb_new = (jax.nn.sigmoid(gb[:, H:2 * H]) * c_b
                  + jax.nn.sigmoid(gb[:, 0:H]) * jnp.tanh(gb[:, 2 * H:3 * H]))
        hb_new = jax.nn.sigmoid(gb[:, 3 * H:4 * H]) * jnp.tanh(cb_new)
        m_b = lens > (t0_b + sb)
        return jnp.where(m_b, hb_new, h_b), jnp.where(m_b, cb_new, c_b)

    def step(k, carry):
        h_f, c_f, h_b, c_b = carry
        s = k * UNROLL
        for u in range(UNROLL):
            h_f, c_f = substep_f(s + u, h_f, c_f)
            h_b, c_b = substep_b(s + u, h_b, c_b)
        return (h_f, c_f, h_b, c_b)

    h_f, c_f, h_b, c_b = jax.lax.fori_loop(
        0, TB // UNROLL, step, (h_f, c_f, h_b, c_b))
    state_ref[:] = jnp.concatenate([h_f, c_f, h_b, c_b], axis=1)

    @pl.when(i == NT - 1)
    def _():
        out_ref[:] = state_ref[:]


def kernel(x, lengths, w_ih_f, w_hh_f, b_ih_f, b_hh_f,
           w_ih_b, w_hh_b, b_ih_b, b_hh_b):
    B, T, D = x.shape
    H = w_hh_f.shape[1]
    TB = 64
    NT = T // TB

    xT = jnp.transpose(x.astype(jnp.bfloat16), (1, 0, 2))
    wif = w_ih_f.T.astype(jnp.bfloat16)
    wib = w_ih_b.T.astype(jnp.bfloat16)
    whf = w_hh_f.T.astype(jnp.bfloat16)
    whb = w_hh_b.T.astype(jnp.bfloat16)
    bf = (b_ih_f + b_hh_f).reshape(1, 4 * H)
    bb = (b_ih_b + b_hh_b).reshape(1, 4 * H)
    lenb = jnp.broadcast_to(lengths.astype(jnp.int32)[:, None], (B, H))

    return pl.pallas_call(
        functools.partial(_bilstm_kernel, TB=TB, NT=NT, B=B, H=H),
        grid=(NT,),
        in_specs=[
            pl.BlockSpec((TB, B, D), lambda i: (i, 0, 0)),
            pl.BlockSpec((TB, B, D), lambda i: (NT - 1 - i, 0, 0)),
            pl.BlockSpec((D, 4 * H), lambda i: (0, 0)),
            pl.BlockSpec((D, 4 * H), lambda i: (0, 0)),
            pl.BlockSpec((H, 4 * H), lambda i: (0, 0)),
            pl.BlockSpec((H, 4 * H), lambda i: (0, 0)),
            pl.BlockSpec((1, 4 * H), lambda i: (0, 0)),
            pl.BlockSpec((1, 4 * H), lambda i: (0, 0)),
            pl.BlockSpec((B, H), lambda i: (0, 0)),
        ],
        out_specs=pl.BlockSpec((B, 4 * H), lambda i: (0, 0)),
        out_shape=jax.ShapeDtypeStruct((B, 4 * H), jnp.float32),
        scratch_shapes=[
            pltpu.VMEM((TB * B, 4 * H), jnp.float32),
            pltpu.VMEM((TB * B, 4 * H), jnp.float32),
            pltpu.VMEM((B, 4 * H), jnp.float32),
        ],
        compiler_params=pltpu.CompilerParams(
            dimension_semantics=("arbitrary",),
        ),
    )(xT, xT, wif, wib, whf, whb, bf, bb, lenb)


# speedup vs baseline: 1.6734x; 1.0300x over previous
import functools

import jax
import jax.numpy as jnp
from jax.experimental import pallas as pl
from jax.experimental.pallas import tpu as pltpu


def _bilstm_kernel(xf_ref, xb_ref, wif_ref, wib_ref, whf_ref, whb_ref,
                   bf_ref, bb_ref, len_ref, out_ref,
                   xgf_ref, xgb_ref, state_ref, *, TB, NT, B, H):
    i = pl.program_id(0)

    xf = xf_ref[:].reshape(TB * B, -1)
    xgf_ref[:] = (
        jnp.dot(xf, wif_ref[:], preferred_element_type=jnp.float32) + bf_ref[:]
    )
    xb = xb_ref[:].reshape(TB * B, -1)
    xgb_ref[:] = (
        jnp.dot(xb, wib_ref[:], preferred_element_type=jnp.float32) + bb_ref[:]
    )

    @pl.when(i == 0)
    def _():
        state_ref[:] = jnp.zeros_like(state_ref)

    h_f = state_ref[:, 0:H]
    c_f = state_ref[:, H:2 * H]
    h_b = state_ref[:, 2 * H:3 * H]
    c_b = state_ref[:, 3 * H:4 * H]

    lens = len_ref[:]
    whf = whf_ref[:]
    whb = whb_ref[:]
    t0_f = i * TB
    t0_b = (NT - 1 - i) * TB

    UNROLL = 16

    def substep_f(s, h_f, c_f):
        gf = xgf_ref[pl.ds(s * B, B), :] + jnp.dot(
            h_f.astype(jnp.bfloat16), whf, preferred_element_type=jnp.float32)
        cf_new = (jax.nn.sigmoid(gf[:, H:2 * H]) * c_f
                  + jax.nn.sigmoid(gf[:, 0:H]) * jnp.tanh(gf[:, 2 * H:3 * H]))
        hf_new = jax.nn.sigmoid(gf[:, 3 * H:4 * H]) * jnp.tanh(cf_new)
        m_f = lens > (t0_f + s)
        return jnp.where(m_f, hf_new, h_f), jnp.where(m_f, cf_new, c_f)

    def substep_b(s, h_b, c_b):
        sb = TB - 1 - s
        gb = xgb_ref[pl.ds(sb * B, B), :] + jnp.dot(
            h_b.astype(jnp.bfloat16), whb, preferred_element_type=jnp.float32)
        cb_new = (jax.nn.sigmoid(gb[:, H:2 * H]) * c_b
                  + jax.nn.sigmoid(gb[:, 0:H]) * jnp.tanh(gb[:, 2 * H:3 * H]))
        hb_new = jax.nn.sigmoid(gb[:, 3 * H:4 * H]) * jnp.tanh(cb_new)
        m_b = lens > (t0_b + sb)
        return jnp.where(m_b, hb_new, h_b), jnp.where(m_b, cb_new, c_b)

    def step(k, carry):
        h_f, c_f, h_b, c_b = carry
        s = k * UNROLL
        for u in range(UNROLL):
            h_f, c_f = substep_f(s + u, h_f, c_f)
            h_b, c_b = substep_b(s + u, h_b, c_b)
        return (h_f, c_f, h_b, c_b)

    h_f, c_f, h_b, c_b = jax.lax.fori_loop(
        0, TB // UNROLL, step, (h_f, c_f, h_b, c_b))
    state_ref[:] = jnp.concatenate([h_f, c_f, h_b, c_b], axis=1)

    @pl.when(i == NT - 1)
    def _():
        out_ref[:] = state_ref[:]


def kernel(x, lengths, w_ih_f, w_hh_f, b_ih_f, b_hh_f,
           w_ih_b, w_hh_b, b_ih_b, b_hh_b):
    B, T, D = x.shape
    H = w_hh_f.shape[1]
    TB = 64
    NT = T // TB

    xT = jnp.transpose(x.astype(jnp.bfloat16), (1, 0, 2))
    wif = w_ih_f.T.astype(jnp.bfloat16)
    wib = w_ih_b.T.astype(jnp.bfloat16)
    whf = w_hh_f.T.astype(jnp.bfloat16)
    whb = w_hh_b.T.astype(jnp.bfloat16)
    bf = (b_ih_f + b_hh_f).reshape(1, 4 * H)
    bb = (b_ih_b + b_hh_b).reshape(1, 4 * H)
    lenb = jnp.broadcast_to(lengths.astype(jnp.int32)[:, None], (B, H))

    return pl.pallas_call(
        functools.partial(_bilstm_kernel, TB=TB, NT=NT, B=B, H=H),
        grid=(NT,),
        in_specs=[
            pl.BlockSpec((TB, B, D), lambda i: (i, 0, 0)),
            pl.BlockSpec((TB, B, D), lambda i: (NT - 1 - i, 0, 0)),
            pl.BlockSpec((D, 4 * H), lambda i: (0, 0)),
            pl.BlockSpec((D, 4 * H), lambda i: (0, 0)),
            pl.BlockSpec((H, 4 * H), lambda i: (0, 0)),
            pl.BlockSpec((H, 4 * H), lambda i: (0, 0)),
            pl.BlockSpec((1, 4 * H), lambda i: (0, 0)),
            pl.BlockSpec((1, 4 * H), lambda i: (0, 0)),
            pl.BlockSpec((B, H), lambda i: (0, 0)),
        ],
        out_specs=pl.BlockSpec((B, 4 * H), lambda i: (0, 0)),
        out_shape=jax.ShapeDtypeStruct((B, 4 * H), jnp.float32),
        scratch_shapes=[
            pltpu.VMEM((TB * B, 4 * H), jnp.float32),
            pltpu.VMEM((TB * B, 4 * H), jnp.float32),
            pltpu.VMEM((B, 4 * H), jnp.float32),
        ],
        compiler_params=pltpu.CompilerParams(
            dimension_semantics=("arbitrary",),
        ),
    )(xT, xT, wif, wib, whf, whb, bf, bb, lenb)
